# baseline (speedup 1.0000x reference)
import functools

import jax
import jax.numpy as jnp
from jax import lax
from jax.experimental import pallas as pl
from jax.experimental.pallas import tpu as pltpu
from jax.experimental.pallas import tpu_sc as plsc

_NC = 2
_NS = 16
_L = 16

_WIDTH = 4
_NW = 13
_CMAX = 1920
_CTAIL = 400
_NCHUNK = 5
_ROWGRP = 8

_SC_ROWS = 256


def _compute_rows(wb, xb, zb, n_lanesteps):
    def lane_body(i, carry):
        s = pl.ds(i * _L, _L)
        w1v = [wb[j, s] for j in range(_WIDTH)]
        b1v = [wb[_WIDTH + j, s] for j in range(_WIDTH)]
        w2v = [wb[2 * _WIDTH + j, s] for j in range(_WIDTH)]
        b2v = wb[3 * _WIDTH, s]
        for r in range(_ROWGRP):
            xv = xb[r, s]
            acc = b2v
            for j in range(_WIDTH):
                h = jnp.maximum(xv * w1v[j] + b1v[j], 0.0)
                acc = acc + h * w2v[j]
            zb[r, s] = acc
        return carry

    lax.fori_loop(0, n_lanesteps, lane_body, 0)


def _sc_body(x_hbm, wpack_hbm, out_hbm,
             wb, xb0, xb1, zb0, zb1, wbt, xbt, zbt,
             wsem, xsem0, xsem1, zsem0, zsem1):
    read_base = x_hbm.shape[0] - _SC_ROWS
    rows_per_worker = _SC_ROWS // (_NC * _NS)

    wid = lax.axis_index("s") * _NC + lax.axis_index("c")
    row_base = wid * rows_per_worker

    xbufs, xsems = (xb0, xb1), (xsem0, xsem1)
    zbufs, zsems = (zb0, zb1), (zsem0, zsem1)

    n_rowgrps = rows_per_worker // _ROWGRP
    items = [(gc, rg) for gc in range(_NCHUNK) for rg in range(n_rowgrps)]
    n_items = len(items)

    def start_w(gc):
        return pltpu.async_copy(
            wpack_hbm.at[:, pl.ds(gc * _CMAX, _CMAX)], wb, wsem)

    def start_x(k):
        gc, rg = items[k]
        r0 = row_base + rg * _ROWGRP
        return pltpu.async_copy(
            x_hbm.at[pl.ds(read_base + r0, _ROWGRP), pl.ds(gc * _CMAX, _CMAX)],
            xbufs[k % 2], xsems[k % 2])

    def start_z(k):
        gc, rg = items[k]
        r0 = row_base + rg * _ROWGRP
        return pltpu.async_copy(
            zbufs[k % 2],
            out_hbm.at[pl.ds(r0, _ROWGRP), pl.ds(gc * _CMAX, _CMAX)],
            zsems[k % 2])

    w_handles = {0: start_w(0)}
    x_handles = {0: start_x(0)}
    z_handles = {}

    for k in range(n_items):
        gc, rg = items[k]
        if k + 1 < n_items:
            x_handles[k + 1] = start_x(k + 1)
        if rg == 0:
            w_handles[gc].wait()
        x_handles[k].wait()
        if k >= 2:
            z_handles[k - 2].wait()
        _compute_rows(wb, xbufs[k % 2], zbufs[k % 2], _CMAX // _L)
        if k + 1 < n_items and items[k + 1][1] == 0:
            w_handles[items[k + 1][0]] = start_w(items[k + 1][0])
        z_handles[k] = start_z(k)
    if n_items >= 2:
        z_handles[n_items - 2].wait()
    z_handles[n_items - 1].wait()

    g0 = _NCHUNK * _CMAX
    pltpu.sync_copy(wpack_hbm.at[:, pl.ds(g0, _CTAIL)], wbt)

    def tail_body(rg, carry):
        r0 = row_base + rg * _ROWGRP
        pltpu.sync_copy(
            x_hbm.at[pl.ds(read_base + r0, _ROWGRP), pl.ds(g0, _CTAIL)], xbt)
        _compute_rows(wbt, xbt, zbt, _CTAIL // _L)
        pltpu.sync_copy(zbt, out_hbm.at[pl.ds(r0, _ROWGRP), pl.ds(g0, _CTAIL)])
        return carry

    lax.fori_loop(0, n_rowgrps, tail_body, 0)


def _tc_body(n_compute, x_ref, w_ref, z_ref, o_ref):
    i = pl.program_id(0)

    @pl.when(i < n_compute)
    def _compute():
        x = x_ref[...]
        width = (w_ref.shape[0] - 1) // 3
        acc = jnp.broadcast_to(w_ref[3 * width : 3 * width + 1, :], x.shape)
        for j in range(width):
            h = jnp.maximum(
                x * w_ref[j : j + 1, :] + w_ref[width + j : width + j + 1, :],
                0.0,
            )
            acc = acc + h * w_ref[2 * width + j : 2 * width + j + 1, :]
        o_ref[...] = acc

    @pl.when(i >= n_compute)
    def _passthrough():
        o_ref[...] = z_ref[...]


def kernel(features, w1, b1, w2, b2, rows1, cols1, rows2, cols2):
    del rows1, cols1, rows2, cols2
    batch, n_genes = features.shape
    width = w1.shape[0] // n_genes
    wpack = jnp.concatenate(
        [
            w1.reshape(n_genes, width).T,
            b1.reshape(n_genes, width).T,
            w2.reshape(n_genes, width).T,
            b2.reshape(1, n_genes),
        ],
        axis=0,
    )

    mesh = plsc.VectorSubcoreMesh(
        core_axis_name="c", subcore_axis_name="s", num_cores=_NC, num_subcores=_NS
    )
    sc_run = functools.partial(
        pl.kernel,
        out_type=jax.ShapeDtypeStruct((_SC_ROWS, n_genes), features.dtype),
        mesh=mesh,
        scratch_types=[
            pltpu.VMEM((_NW, _CMAX), jnp.float32),
            pltpu.VMEM((_ROWGRP, _CMAX), jnp.float32),
            pltpu.VMEM((_ROWGRP, _CMAX), jnp.float32),
            pltpu.VMEM((_ROWGRP, _CMAX), jnp.float32),
            pltpu.VMEM((_ROWGRP, _CMAX), jnp.float32),
            pltpu.VMEM((_NW, _CTAIL), jnp.float32),
            pltpu.VMEM((_ROWGRP, _CTAIL), jnp.float32),
            pltpu.VMEM((_ROWGRP, _CTAIL), jnp.float32),
            pltpu.SemaphoreType.DMA,
            pltpu.SemaphoreType.DMA,
            pltpu.SemaphoreType.DMA,
            pltpu.SemaphoreType.DMA,
            pltpu.SemaphoreType.DMA,
        ],
    )(_sc_body)
    z_sc = sc_run(features, wpack)

    bt = 128
    n_compute = (batch - _SC_ROWS) // bt
    n_pass = _SC_ROWS // bt
    return pl.pallas_call(
        functools.partial(_tc_body, n_compute),
        grid=(n_compute + n_pass,),
        in_specs=[
            pl.BlockSpec(
                (bt, n_genes), lambda i: (jnp.minimum(i, n_compute - 1), 0)
            ),
            pl.BlockSpec((3 * width + 1, n_genes), lambda i: (0, 0)),
            pl.BlockSpec(
                (bt, n_genes),
                lambda i: (jnp.maximum(i - n_compute, 0), 0),
            ),
        ],
        out_specs=pl.BlockSpec((bt, n_genes), lambda i: (i, 0)),
        out_shape=jax.ShapeDtypeStruct((batch, n_genes), features.dtype),
    )(features, wpack, z_sc)


# speedup vs baseline: 1.1583x; 1.1583x over previous
import functools

import jax
import jax.numpy as jnp
from jax import lax
from jax.experimental import pallas as pl
from jax.experimental.pallas import tpu as pltpu
from jax.experimental.pallas import tpu_sc as plsc

_NC = 2
_NS = 16
_L = 16

_WIDTH = 4
_NW = 13
_CMAX = 1920
_CTAIL = 400
_NCHUNK = 5
_ROWGRP = 8

_SC_ROWS = 256


def _compute_rows(wb, xb, zb, n_lanesteps):
    def lane_body(i, carry):
        s = pl.ds(i * _L, _L)
        w1v = [wb[j, s] for j in range(_WIDTH)]
        b1v = [wb[_WIDTH + j, s] for j in range(_WIDTH)]
        w2v = [wb[2 * _WIDTH + j, s] for j in range(_WIDTH)]
        b2v = wb[3 * _WIDTH, s]
        for r in range(_ROWGRP):
            xv = xb[r, s]
            acc = b2v
            for j in range(_WIDTH):
                h = jnp.maximum(xv * w1v[j] + b1v[j], 0.0)
                acc = acc + h * w2v[j]
            zb[r, s] = acc
        return carry

    lax.fori_loop(0, n_lanesteps, lane_body, 0)


def _sc_body(x_hbm, wpack_hbm, out_hbm,
             wb, xb0, xb1, zb0, zb1, wbt, xbt, zbt,
             wsem, xsem0, xsem1, zsem0, zsem1):
    read_base = x_hbm.shape[0] - _SC_ROWS
    rows_per_worker = _SC_ROWS // (_NC * _NS)

    wid = lax.axis_index("s") * _NC + lax.axis_index("c")
    row_base = wid * rows_per_worker

    xbufs, xsems = (xb0, xb1), (xsem0, xsem1)
    zbufs, zsems = (zb0, zb1), (zsem0, zsem1)

    n_rowgrps = rows_per_worker // _ROWGRP
    items = [(gc, rg) for gc in range(_NCHUNK) for rg in range(n_rowgrps)]
    n_items = len(items)

    def start_w(gc):
        return pltpu.async_copy(
            wpack_hbm.at[:, pl.ds(gc * _CMAX, _CMAX)], wb, wsem)

    def start_x(k):
        gc, rg = items[k]
        r0 = read_base + row_base + rg * _ROWGRP
        return pltpu.async_copy(
            x_hbm.at[pl.ds(r0, _ROWGRP), pl.ds(gc * _CMAX, _CMAX)],
            xbufs[k % 2], xsems[k % 2])

    def start_z(k):
        gc, rg = items[k]
        r0 = read_base + row_base + rg * _ROWGRP
        return pltpu.async_copy(
            zbufs[k % 2],
            out_hbm.at[pl.ds(r0, _ROWGRP), pl.ds(gc * _CMAX, _CMAX)],
            zsems[k % 2])

    w_handles = {0: start_w(0)}
    x_handles = {0: start_x(0)}
    z_handles = {}

    for k in range(n_items):
        gc, rg = items[k]
        if k + 1 < n_items:
            x_handles[k + 1] = start_x(k + 1)
        if rg == 0:
            w_handles[gc].wait()
        x_handles[k].wait()
        if k >= 2:
            z_handles[k - 2].wait()
        _compute_rows(wb, xbufs[k % 2], zbufs[k % 2], _CMAX // _L)
        if k + 1 < n_items and items[k + 1][1] == 0:
            w_handles[items[k + 1][0]] = start_w(items[k + 1][0])
        z_handles[k] = start_z(k)
    if n_items >= 2:
        z_handles[n_items - 2].wait()
    z_handles[n_items - 1].wait()

    g0 = _NCHUNK * _CMAX
    tailw = wpack_hbm.shape[1] - g0
    pltpu.sync_copy(wpack_hbm.at[:, pl.ds(g0, tailw)], wbt)

    def tail_body(rg, carry):
        r0 = read_base + row_base + rg * _ROWGRP
        pltpu.sync_copy(x_hbm.at[pl.ds(r0, _ROWGRP), pl.ds(g0, _CTAIL)], xbt)
        _compute_rows(wbt, xbt, zbt, _CTAIL // _L)
        pltpu.sync_copy(zbt, out_hbm.at[pl.ds(r0, _ROWGRP), pl.ds(g0, _CTAIL)])
        return carry

    lax.fori_loop(0, n_rowgrps, tail_body, 0)


def _tc_body(n_genes, x_ref, w_ref, z_ref, o_ref):
    del z_ref
    x = x_ref[...]
    width = (w_ref.shape[0] - 1) // 3
    acc = jnp.broadcast_to(w_ref[3 * width : 3 * width + 1, :n_genes], x.shape)
    for j in range(width):
        h = jnp.maximum(
            x * w_ref[j : j + 1, :n_genes]
            + w_ref[width + j : width + j + 1, :n_genes],
            0.0,
        )
        acc = acc + h * w_ref[2 * width + j : 2 * width + j + 1, :n_genes]
    o_ref[...] = acc


def _pack_weights(w1, b1, w2, b2, n_genes, width):
    lanes = width * 128
    n_pad = -(-(width * n_genes) // lanes) * lanes
    g_pad = n_pad // width
    wstack = jnp.stack([w1, b1, w2])
    wstack = jnp.pad(wstack, ((0, 0), (0, n_pad - width * n_genes)))
    wmat = wstack.reshape(3, n_pad // lanes, lanes)
    c_i = jax.lax.broadcasted_iota(jnp.int32, (lanes, lanes), 0)
    m_i = jax.lax.broadcasted_iota(jnp.int32, (lanes, lanes), 1)
    sel = (c_i == width * (m_i % 128) + m_i // 128).astype(jnp.float32)
    deint = jax.lax.dot_general(
        wmat, sel, (((2,), (0,)), ((), ())),
        precision=jax.lax.Precision.HIGHEST,
    )
    deint = (deint.reshape(3, n_pad // lanes, width, 128)
             .transpose(0, 2, 1, 3).reshape(3 * width, g_pad))
    b2p = jnp.pad(b2, (0, g_pad - n_genes)).reshape(1, g_pad)
    return jnp.concatenate([deint, b2p], axis=0)


def kernel(features, w1, b1, w2, b2, rows1, cols1, rows2, cols2):
    del rows1, cols1, rows2, cols2
    batch, n_genes = features.shape
    width = w1.shape[0] // n_genes
    wpack = _pack_weights(w1, b1, w2, b2, n_genes, width)

    mesh = plsc.VectorSubcoreMesh(
        core_axis_name="c", subcore_axis_name="s", num_cores=_NC, num_subcores=_NS
    )
    sc_run = functools.partial(
        pl.kernel,
        out_type=jax.ShapeDtypeStruct((batch, n_genes), features.dtype),
        mesh=mesh,
        scratch_types=[
            pltpu.VMEM((_NW, _CMAX), jnp.float32),
            pltpu.VMEM((_ROWGRP, _CMAX), jnp.float32),
            pltpu.VMEM((_ROWGRP, _CMAX), jnp.float32),
            pltpu.VMEM((_ROWGRP, _CMAX), jnp.float32),
            pltpu.VMEM((_ROWGRP, _CMAX), jnp.float32),
            pltpu.VMEM((_NW, wpack.shape[1] - _NCHUNK * _CMAX), jnp.float32),
            pltpu.VMEM((_ROWGRP, _CTAIL), jnp.float32),
            pltpu.VMEM((_ROWGRP, _CTAIL), jnp.float32),
            pltpu.SemaphoreType.DMA,
            pltpu.SemaphoreType.DMA,
            pltpu.SemaphoreType.DMA,
            pltpu.SemaphoreType.DMA,
            pltpu.SemaphoreType.DMA,
        ],
    )(_sc_body)
    z0 = sc_run(features, wpack)

    bt = 128
    n_compute = (batch - _SC_ROWS) // bt
    g_pad = wpack.shape[1]
    return pl.pallas_call(
        functools.partial(_tc_body, n_genes),
        grid=(n_compute,),
        in_specs=[
            pl.BlockSpec((bt, n_genes), lambda i: (i, 0)),
            pl.BlockSpec((_NW, g_pad), lambda i: (0, 0)),
            pl.BlockSpec(memory_space=pl.ANY),
        ],
        out_specs=pl.BlockSpec((bt, n_genes), lambda i: (i, 0)),
        out_shape=jax.ShapeDtypeStruct((batch, n_genes), features.dtype),
        input_output_aliases={2: 0},
    )(features, wpack, z0)


# speedup vs baseline: 1.2773x; 1.1028x over previous
import functools

import jax
import jax.numpy as jnp
from jax import lax
from jax.experimental import pallas as pl
from jax.experimental.pallas import tpu as pltpu
from jax.experimental.pallas import tpu_sc as plsc

_NC = 2
_NS = 16
_L = 16

_WIDTH = 4
_NW = 13
_CMAX = 1920
_CTAIL = 400
_NCHUNK = 5
_ROWGRP = 8

_SC_ROWS = 256


def _compute_rows(wb, xb, zb, n_lanesteps):
    def lane_body(i, carry):
        s = pl.ds(i * _L, _L)
        w1v = [wb[j, s] for j in range(_WIDTH)]
        b1v = [wb[_WIDTH + j, s] for j in range(_WIDTH)]
        w2v = [wb[2 * _WIDTH + j, s] for j in range(_WIDTH)]
        b2v = wb[3 * _WIDTH, s]
        for r in range(_ROWGRP):
            xv = xb[r, s]
            acc = b2v
            for j in range(_WIDTH):
                h = jnp.maximum(xv * w1v[j] + b1v[j], 0.0)
                acc = acc + h * w2v[j]
            zb[r, s] = acc
        return carry

    lax.fori_loop(0, n_lanesteps, lane_body, 0)


def _sc_body(x_hbm, wpack_hbm, out_hbm,
             wb, xb0, xb1, zb0, zb1, wbt, xbt, zbt,
             wsem, xsem0, xsem1, zsem0, zsem1):
    read_base = x_hbm.shape[0] - _SC_ROWS
    rows_per_worker = _SC_ROWS // (_NC * _NS)

    wid = lax.axis_index("s") * _NC + lax.axis_index("c")
    row_base = wid * rows_per_worker

    xbufs, xsems = (xb0, xb1), (xsem0, xsem1)
    zbufs, zsems = (zb0, zb1), (zsem0, zsem1)

    n_rowgrps = rows_per_worker // _ROWGRP
    items = [(gc, rg) for gc in range(_NCHUNK) for rg in range(n_rowgrps)]
    n_items = len(items)

    def start_w(gc):
        return pltpu.async_copy(
            wpack_hbm.at[:, pl.ds(gc * _CMAX, _CMAX)], wb, wsem)

    def start_x(k):
        gc, rg = items[k]
        r0 = read_base + row_base + rg * _ROWGRP
        return pltpu.async_copy(
            x_hbm.at[pl.ds(r0, _ROWGRP), pl.ds(gc * _CMAX, _CMAX)],
            xbufs[k % 2], xsems[k % 2])

    def start_z(k):
        gc, rg = items[k]
        r0 = row_base + rg * _ROWGRP
        return pltpu.async_copy(
            zbufs[k % 2],
            out_hbm.at[pl.ds(r0, _ROWGRP), pl.ds(gc * _CMAX, _CMAX)],
            zsems[k % 2])

    w_handles = {0: start_w(0)}
    x_handles = {0: start_x(0)}
    z_handles = {}

    for k in range(n_items):
        gc, rg = items[k]
        if k + 1 < n_items:
            x_handles[k + 1] = start_x(k + 1)
        if rg == 0:
            w_handles[gc].wait()
        x_handles[k].wait()
        if k >= 2:
            z_handles[k - 2].wait()
        _compute_rows(wb, xbufs[k % 2], zbufs[k % 2], _CMAX // _L)
        if k + 1 < n_items and items[k + 1][1] == 0:
            w_handles[items[k + 1][0]] = start_w(items[k + 1][0])
        z_handles[k] = start_z(k)
    if n_items >= 2:
        z_handles[n_items - 2].wait()
    z_handles[n_items - 1].wait()

    g0 = _NCHUNK * _CMAX
    tailw = wpack_hbm.shape[1] - g0
    pltpu.sync_copy(wpack_hbm.at[:, pl.ds(g0, tailw)], wbt)

    def tail_body(rg, carry):
        r0 = row_base + rg * _ROWGRP
        pltpu.sync_copy(
            x_hbm.at[pl.ds(read_base + r0, _ROWGRP), pl.ds(g0, _CTAIL)], xbt)
        _compute_rows(wbt, xbt, zbt, _CTAIL // _L)
        pltpu.sync_copy(zbt, out_hbm.at[pl.ds(r0, _ROWGRP), pl.ds(g0, _CTAIL)])
        return carry

    lax.fori_loop(0, n_rowgrps, tail_body, 0)


def _tc_body(n_genes, x_ref, w_ref, o_ref):
    x = x_ref[...]
    width = (w_ref.shape[0] - 1) // 3
    acc = jnp.broadcast_to(w_ref[3 * width : 3 * width + 1, :n_genes], x.shape)
    for j in range(width):
        h = jnp.maximum(
            x * w_ref[j : j + 1, :n_genes]
            + w_ref[width + j : width + j + 1, :n_genes],
            0.0,
        )
        acc = acc + h * w_ref[2 * width + j : 2 * width + j + 1, :n_genes]
    o_ref[...] = acc


def _pack_weights(w1, b1, w2, b2, n_genes, width):
    lanes = width * 128
    n_pad = -(-(width * n_genes) // lanes) * lanes
    g_pad = n_pad // width
    wstack = jnp.stack([w1, b1, w2])
    wstack = jnp.pad(wstack, ((0, 0), (0, n_pad - width * n_genes)))
    wmat = wstack.reshape(3, n_pad // lanes, lanes)
    c_i = jax.lax.broadcasted_iota(jnp.int32, (lanes, lanes), 0)
    m_i = jax.lax.broadcasted_iota(jnp.int32, (lanes, lanes), 1)
    sel = (c_i == width * (m_i % 128) + m_i // 128).astype(jnp.float32)
    deint = jax.lax.dot_general(
        wmat, sel, (((2,), (0,)), ((), ())),
        precision=jax.lax.Precision.HIGHEST,
    )
    deint = (deint.reshape(3, n_pad // lanes, width, 128)
             .transpose(0, 2, 1, 3).reshape(3 * width, g_pad))
    b2p = jnp.pad(b2, (0, g_pad - n_genes)).reshape(1, g_pad)
    return jnp.concatenate([deint, b2p], axis=0)


def kernel(features, w1, b1, w2, b2, rows1, cols1, rows2, cols2):
    del rows1, cols1, rows2, cols2
    batch, n_genes = features.shape
    width = w1.shape[0] // n_genes
    wpack = _pack_weights(w1, b1, w2, b2, n_genes, width)

    mesh = plsc.VectorSubcoreMesh(
        core_axis_name="c", subcore_axis_name="s", num_cores=_NC, num_subcores=_NS
    )
    sc_run = functools.partial(
        pl.kernel,
        out_type=jax.ShapeDtypeStruct((_SC_ROWS, n_genes), features.dtype),
        mesh=mesh,
        scratch_types=[
            pltpu.VMEM((_NW, _CMAX), jnp.float32),
            pltpu.VMEM((_ROWGRP, _CMAX), jnp.float32),
            pltpu.VMEM((_ROWGRP, _CMAX), jnp.float32),
            pltpu.VMEM((_ROWGRP, _CMAX), jnp.float32),
            pltpu.VMEM((_ROWGRP, _CMAX), jnp.float32),
            pltpu.VMEM((_NW, wpack.shape[1] - _NCHUNK * _CMAX), jnp.float32),
            pltpu.VMEM((_ROWGRP, _CTAIL), jnp.float32),
            pltpu.VMEM((_ROWGRP, _CTAIL), jnp.float32),
            pltpu.SemaphoreType.DMA,
            pltpu.SemaphoreType.DMA,
            pltpu.SemaphoreType.DMA,
            pltpu.SemaphoreType.DMA,
            pltpu.SemaphoreType.DMA,
        ],
    )(_sc_body)
    z_sc = sc_run(features, wpack)

    bt = 128
    n_compute = (batch - _SC_ROWS) // bt
    g_pad = wpack.shape[1]
    z_tc = pl.pallas_call(
        functools.partial(_tc_body, n_genes),
        grid=(n_compute,),
        in_specs=[
            pl.BlockSpec((bt, n_genes), lambda i: (i, 0)),
            pl.BlockSpec((_NW, g_pad), lambda i: (0, 0)),
        ],
        out_specs=pl.BlockSpec((bt, n_genes), lambda i: (i, 0)),
        out_shape=jax.ShapeDtypeStruct((batch, n_genes), features.dtype),
    )(features, wpack)
    return lax.dynamic_update_slice(z_tc, z_sc, (batch - _SC_ROWS, 0))
